# baseline (speedup 1.0000x reference)
import jax
import jax.numpy as jnp
from jax import lax
from jax.experimental import pallas as pl
from jax.experimental.pallas import tpu as pltpu
from jax.experimental.pallas import tpu_sc as plsc


def _copy_body(src_ref, dst_ref):
    dst_ref[...] = src_ref[...]


def _tc_copy(a, block_rows):
    rows, cols = a.shape
    assert rows % block_rows == 0
    return pl.pallas_call(
        _copy_body,
        grid=(rows // block_rows,),
        in_specs=[pl.BlockSpec((block_rows, cols), lambda i: (i, 0))],
        out_specs=pl.BlockSpec((block_rows, cols), lambda i: (i, 0)),
        out_shape=jax.ShapeDtypeStruct(a.shape, a.dtype),
    )(a)


def _sc_copy(a):
    rows, cols = a.shape
    n_workers = 32
    stripe = (rows // (64 * n_workers)) * 64
    tail = rows - stripe * n_workers
    assert tail % 64 == 0
    tail_per = 64
    n_tail_workers = tail // tail_per
    tail_base = stripe * n_workers
    mesh = plsc.VectorSubcoreMesh(core_axis_name="c", subcore_axis_name="s")

    def body(src_hbm, dst_hbm):
        wid = lax.axis_index("s") * 2 + lax.axis_index("c")
        base = wid * stripe
        pltpu.sync_copy(
            src_hbm.at[pl.ds(base, stripe), :],
            dst_hbm.at[pl.ds(base, stripe), :],
        )

        @pl.when(wid < n_tail_workers)
        def _():
            tb = tail_base + wid * tail_per
            pltpu.sync_copy(
                src_hbm.at[pl.ds(tb, tail_per), :],
                dst_hbm.at[pl.ds(tb, tail_per), :],
            )

    return pl.kernel(
        body,
        mesh=mesh,
        out_type=jax.ShapeDtypeStruct(a.shape, a.dtype),
    )(a)


def kernel(x, edge_index, edge_attr):
    del edge_index
    x_out = _tc_copy(x, 1000)
    ea_out = _sc_copy(edge_attr)
    return (x_out, ea_out)


# speedup vs baseline: 18.4806x; 18.4806x over previous
import jax
import jax.numpy as jnp
from jax import lax
from jax.experimental import pallas as pl
from jax.experimental.pallas import tpu as pltpu
from jax.experimental.pallas import tpu_sc as plsc


def _copy_body(src_ref, dst_ref):
    dst_ref[...] = src_ref[...]


def _tc_copy(a, block_rows):
    rows, cols = a.shape
    assert rows % block_rows == 0
    return pl.pallas_call(
        _copy_body,
        grid=(rows // block_rows,),
        in_specs=[pl.BlockSpec((block_rows, cols), lambda i: (i, 0))],
        out_specs=pl.BlockSpec((block_rows, cols), lambda i: (i, 0)),
        out_shape=jax.ShapeDtypeStruct(a.shape, a.dtype),
    )(a)


def kernel(x, edge_index, edge_attr):
    del edge_index
    x_out = _tc_copy(x, 1000)
    ea_out = _tc_copy(edge_attr, 16000)
    return (x_out, ea_out)
